# baseline (speedup 1.0000x reference)
import functools

import jax
import jax.numpy as jnp
from jax import lax
from jax.experimental import pallas as pl
from jax.experimental.pallas import tpu as pltpu
from jax.experimental.pallas import tpu_sc as plsc

B = 4
T = 2048
D = 1024
L = 16
NC = 2
NS = 16
NW = NC * NS
PER = (B * T) // NW
VPT = PER // L
VPR = T // L
R = 16
NCH = PER // R
NB = 6
NF = 4


def _body(inp_hbm, tab_hbm, out_hbm, row_v, chunk_v, idx_v, buf_v, *sems):
    wid = lax.axis_index("c") * NS + lax.axis_index("s")
    row = wid // (T // PER)
    ch = wid % (T // PER)

    c_row = pltpu.async_copy(inp_hbm.at[pl.ds(row * T, T)], row_v, sems[0])
    c_chk = pltpu.async_copy(inp_hbm.at[pl.ds(wid * PER, PER)], chunk_v, sems[1])
    c_row.wait()
    c_chk.wait()

    zeros = jnp.zeros((L,), jnp.int32)
    lane = lax.iota(jnp.int32, L)
    last = jnp.broadcast_to(jnp.int32(L - 1), (L,))

    def nonzero_mask(v):
        return lax.shift_right_logical(v | (zeros - v), 31)

    def scan16(x):
        s = x
        for d in (1, 2, 4, 8):
            shifted = s.at[jnp.maximum(lane - d, 0)].get(mode="promise_in_bounds")
            s = s + jnp.clip(lane - (d - 1), 0, 1) * shifted
        return s

    def splat_last(s):
        return s.at[last].get(mode="promise_in_bounds")

    chvec = jnp.broadcast_to(ch, (L,))
    sumvec = zeros
    for c2 in range(T // PER):
        csum = zeros
        for j in range(VPT):
            csum = csum + nonzero_mask(row_v[pl.ds((c2 * VPT + j) * L, L)])
        sumvec = sumvec + jnp.clip(chvec - c2, 0, 1) * csum
    prefix = splat_last(scan16(sumvec))

    carry = prefix
    for j in range(VPT):
        v = chunk_v[pl.ds(j * L, L)]
        mi = nonzero_mask(v)
        s = scan16(mi)
        pos = (carry + s) * mi
        idx_v[j // (R // L), pl.ds((j % (R // L)) * L, L)] = pos
        carry = carry + splat_last(s)

    base = wid * PER
    gsems = sems[:NB]
    osems = sems[NB:]

    def gather(k):
        return pltpu.async_copy(
            tab_hbm.at[idx_v.at[(k + row * 4) % NCH]],
            buf_v.at[pl.ds((k % NB) * R, R)], gsems[k % NB])

    def outpair(m):
        return pltpu.async_copy(
            buf_v.at[pl.ds(((2 * m) % NB) * R, 2 * R)],
            out_hbm.at[pl.ds(base + ((2 * m + row * 4) % NCH) * R, 2 * R)],
            osems[m % (NB // 2)])

    gs = [gather(k) for k in range(min(NF, NCH))]
    os_ = []
    for k in range(NCH):
        gs[k].wait()
        if k % 2 == 1:
            os_.append(outpair((k - 1) // 2))
        nxt = k + NF
        if nxt < NCH:
            if nxt - NB >= 0 and (nxt - NB) % 2 == 0:
                os_[(nxt - NB) // 2].wait()
            gs.append(gather(nxt))
    for m in range(max(0, (NCH - NB) // 2), NCH // 2):
        os_[m].wait()


@jax.jit
def _sc_embed(flat_inp, table):
    mesh = plsc.VectorSubcoreMesh(
        core_axis_name="c", subcore_axis_name="s", num_cores=NC, num_subcores=NS
    )
    return pl.kernel(
        _body,
        out_type=jax.ShapeDtypeStruct((B * T, D), jnp.float32),
        mesh=mesh,
        scratch_types=[
            pltpu.VMEM((T,), jnp.int32),
            pltpu.VMEM((PER,), jnp.int32),
            pltpu.VMEM((NCH, R), jnp.int32),
            pltpu.VMEM((NB * R, D), jnp.float32),
        ] + [pltpu.SemaphoreType.DMA] * (NB + NB // 2),
    )(flat_inp, table)


def kernel(input, embeddings):
    flat = input.reshape(-1).astype(jnp.int32)
    out = _sc_embed(flat, embeddings.astype(jnp.float32))
    return out.reshape(B, T, D)


# speedup vs baseline: 1.0082x; 1.0082x over previous
import jax
import jax.numpy as jnp
from jax import lax
from jax.experimental import pallas as pl
from jax.experimental.pallas import tpu as pltpu
from jax.experimental.pallas import tpu_sc as plsc

B = 4
T = 2048
D = 1024
L = 16
NC = 2
NS = 16
NW = NC * NS
PER = (B * T) // NW
VPT = PER // L
VPR = T // L
R = 16
NCH = PER // R
NB = 6
NF = 5


def _body(inp_hbm, tab_hbm, out_hbm, row_v, chunk_v, idx_v, buf_v, *sems):
    wid = lax.axis_index("c") * NS + lax.axis_index("s")
    row = wid // (T // PER)
    ch = wid % (T // PER)

    c_row = pltpu.async_copy(inp_hbm.at[pl.ds(row * T, T)], row_v, sems[0])
    c_chk = pltpu.async_copy(inp_hbm.at[pl.ds(wid * PER, PER)], chunk_v, sems[1])
    c_row.wait()
    c_chk.wait()

    zeros = jnp.zeros((L,), jnp.int32)
    lane = lax.iota(jnp.int32, L)
    last = jnp.broadcast_to(jnp.int32(L - 1), (L,))

    def nonzero_mask(v):
        return lax.shift_right_logical(v | (zeros - v), 31)

    def scan16(x):
        s = x
        for d in (1, 2, 4, 8):
            shifted = s.at[jnp.maximum(lane - d, 0)].get(mode="promise_in_bounds")
            s = s + jnp.clip(lane - (d - 1), 0, 1) * shifted
        return s

    def splat_last(s):
        return s.at[last].get(mode="promise_in_bounds")

    chvec = jnp.broadcast_to(ch, (L,))
    sumvec = zeros
    for c2 in range(T // PER):
        csum = zeros
        for j in range(VPT):
            csum = csum + nonzero_mask(row_v[pl.ds((c2 * VPT + j) * L, L)])
        sumvec = sumvec + jnp.clip(chvec - c2, 0, 1) * csum
    prefix = splat_last(scan16(sumvec))

    carry = prefix
    for j in range(VPT):
        v = chunk_v[pl.ds(j * L, L)]
        mi = nonzero_mask(v)
        s = scan16(mi)
        pos = (carry + s) * mi
        idx_v[j // (R // L), pl.ds((j % (R // L)) * L, L)] = pos
        carry = carry + splat_last(s)

    base = wid * PER
    gsems = sems[:NB]
    osems = sems[NB:]

    def gather(k):
        return pltpu.async_copy(
            tab_hbm.at[idx_v.at[(k + row * 4) % NCH]],
            buf_v.at[pl.ds((k % NB) * R, R)], gsems[k % NB])

    def outpair(m):
        return pltpu.async_copy(
            buf_v.at[pl.ds(((2 * m) % NB) * R, 2 * R)],
            out_hbm.at[pl.ds(base + ((2 * m + row * 4) % NCH) * R, 2 * R)],
            osems[m % (NB // 2)])

    gs = [gather(k) for k in range(min(NF, NCH))]
    os_ = []
    for k in range(NCH):
        gs[k].wait()
        if k % 2 == 1:
            os_.append(outpair((k - 1) // 2))
        nxt = k + NF
        if nxt < NCH:
            if nxt - NB >= 0 and (nxt - NB) % 2 == 0:
                os_[(nxt - NB) // 2].wait()
            gs.append(gather(nxt))
    for m in range(max(0, (NCH - NB) // 2), NCH // 2):
        os_[m].wait()


@jax.jit
def _sc_embed(flat_inp, table):
    mesh = plsc.VectorSubcoreMesh(
        core_axis_name="c", subcore_axis_name="s", num_cores=NC, num_subcores=NS
    )
    return pl.kernel(
        _body,
        out_type=jax.ShapeDtypeStruct((B * T, D), jnp.float32),
        mesh=mesh,
        scratch_types=[
            pltpu.VMEM((T,), jnp.int32),
            pltpu.VMEM((PER,), jnp.int32),
            pltpu.VMEM((NCH, R), jnp.int32),
            pltpu.VMEM((NB * R, D), jnp.float32),
        ] + [pltpu.SemaphoreType.DMA] * (NB + NB // 2),
    )(flat_inp, table)


def kernel(input, embeddings):
    flat = input.reshape(-1).astype(jnp.int32)
    out = _sc_embed(flat, embeddings.astype(jnp.float32))
    return out.reshape(B, T, D)
